# baseline (speedup 1.0000x reference)
import functools

import jax
import jax.numpy as jnp
from jax import lax
from jax.experimental import pallas as pl
from jax.experimental.pallas import tpu as pltpu
from jax.experimental.pallas import tpu_sc as plsc

EMB = 64
SEQ = 200
SCALE = 8.0

NC = 2
NS = 16
NW = NC * NS
LANES = 16
NREG = EMB // LANES

B_TOT = 4096 * 200
B_PER_W = B_TOT // NW
CHUNK = 128
STEPS = B_PER_W // CHUNK


def _body(idx_hbm, tok_hbm, pos_hbm, out_hbm,
          idx_v, pos_v, ib0, ib1, ob0, ob1, sg0, sg1, ss0, ss1):
    wid = lax.axis_index("s") * NC + lax.axis_index("c")
    base = wid * B_PER_W

    pltpu.sync_copy(idx_hbm.at[wid], idx_v)
    pltpu.sync_copy(pos_hbm, pos_v)

    ibufs = (ib0, ib1)
    obufs = (ob0, ob1)
    sg = (sg0, sg1)
    ss = (ss0, ss1)

    def issue_gather(h, b):
        pltpu.async_copy(tok_hbm.at[idx_v.at[h]], ibufs[b], sg[b])

    def drain_gather(h, b):
        pltpu.make_async_copy(tok_hbm.at[idx_v.at[h]], ibufs[b], sg[b]).wait()

    def issue_scatter(h, b):
        pltpu.async_copy(obufs[b], out_hbm.at[pl.ds(base + h * CHUNK, CHUNK)],
                         ss[b])

    def drain_scatter(h, b):
        pltpu.make_async_copy(
            obufs[b], out_hbm.at[pl.ds(base + h * CHUNK, CHUNK)], ss[b]).wait()

    def compute(h, b):
        ib, ob = ibufs[b], obufs[b]

        def row(r, carry):
            p = lax.rem(h * CHUNK + r, SEQ)
            for e in range(NREG):
                pv = pos_v[p, pl.ds(e * LANES, LANES)]
                tv = ib[r, pl.ds(e * LANES, LANES)]
                ob[r, pl.ds(e * LANES, LANES)] = tv * SCALE + pv
            return carry

        lax.fori_loop(0, CHUNK, row, 0)

    issue_gather(0, 0)
    issue_gather(1, 1)

    def outer(i, carry):
        for b in range(2):
            h = i * 2 + b
            drain_gather(h, b)

            @pl.when(h >= 2)
            def _():
                drain_scatter(h - 2, b)

            compute(h, b)
            issue_scatter(h, b)

            @pl.when(h + 2 < STEPS)
            def _():
                issue_gather(h + 2, b)
        return carry

    lax.fori_loop(0, STEPS // 2, outer, 0)
    drain_scatter(STEPS - 2, 0)
    drain_scatter(STEPS - 1, 1)


@functools.partial(jax.jit, static_argnums=())
def _emb_lookup(idx, token_table, pos_table):
    fn = pl.kernel(
        _body,
        out_type=jax.ShapeDtypeStruct((B_TOT, EMB), jnp.float32),
        mesh=plsc.VectorSubcoreMesh(core_axis_name="c", subcore_axis_name="s"),
        compiler_params=pltpu.CompilerParams(use_tc_tiling_on_sc=False),
        scratch_types=[
            pltpu.VMEM((STEPS, CHUNK), jnp.int32),
            pltpu.VMEM((SEQ, EMB), jnp.float32),
            pltpu.VMEM((CHUNK, EMB), jnp.float32),
            pltpu.VMEM((CHUNK, EMB), jnp.float32),
            pltpu.VMEM((CHUNK, EMB), jnp.float32),
            pltpu.VMEM((CHUNK, EMB), jnp.float32),
            pltpu.SemaphoreType.DMA,
            pltpu.SemaphoreType.DMA,
            pltpu.SemaphoreType.DMA,
            pltpu.SemaphoreType.DMA,
        ],
    )
    return fn(idx, token_table, pos_table)


def kernel(src, tgt, token_table, pos_table):
    del tgt
    B, S = src.shape
    idx = src.reshape(NW, STEPS, CHUNK)
    out = _emb_lookup(idx, token_table, pos_table)
    return out.reshape(B, S, EMB)


# speedup vs baseline: 1.2598x; 1.2598x over previous
import functools

import jax
import jax.numpy as jnp
from jax import lax
from jax.experimental import pallas as pl
from jax.experimental.pallas import tpu as pltpu
from jax.experimental.pallas import tpu_sc as plsc

EMB = 64
SEQ = 200
SCALE = 8.0

NC = 2
NS = 16
NW = NC * NS
LANES = 16
NREG = EMB // LANES

BATCH = 4096
B_PER_W = BATCH // NW
G = 2
NG = B_PER_W // G
HALF = SEQ // 2


def _body(src_hbm, tok_hbm, pos_hbm, out_hbm,
          ix0, ix1, ib0, ib1, ob0, ob1, pos_v,
          si0, si1, sg0, sg1, ss0, ss1):
    wid = lax.axis_index("s") * NC + lax.axis_index("c")
    row0 = wid * B_PER_W

    pltpu.sync_copy(pos_hbm, pos_v)

    ixs = (ix0, ix1)
    ibs = (ib0, ib1)
    obs = (ob0, ob1)
    si = (si0, si1)
    sg = (sg0, sg1)
    ss = (ss0, ss1)

    def issue_idx(g, b):
        for j in range(G):
            pltpu.async_copy(src_hbm.at[row0 + g * G + j], ixs[b].at[j], si[b])

    def drain_idx(g, b):
        for j in range(G):
            pltpu.make_async_copy(
                src_hbm.at[row0 + g * G + j], ixs[b].at[j], si[b]).wait()

    def issue_gather(g, b):
        del g
        for j in range(G):
            pltpu.async_copy(
                tok_hbm.at[ixs[b].at[j]], ibs[b].at[j], sg[b])

    def drain_gather(g, b):
        del g
        for j in range(G):
            pltpu.make_async_copy(
                tok_hbm.at[ixs[b].at[j]], ibs[b].at[j], sg[b]).wait()

    def issue_scatter(g, b):
        pltpu.async_copy(obs[b], out_hbm.at[pl.ds(row0 + g * G, G)], ss[b])

    def drain_scatter(g, b):
        pltpu.make_async_copy(
            obs[b], out_hbm.at[pl.ds(row0 + g * G, G)], ss[b]).wait()

    def compute(b):
        ib, ob = ibs[b], obs[b]

        def row(r, carry):
            for e in range(NREG):
                pv = pos_v[r, pl.ds(e * LANES, LANES)]
                for j in range(G):
                    tv = ib[j, r, pl.ds(e * LANES, LANES)]
                    ob[j, r, pl.ds(e * LANES, LANES)] = tv * SCALE + pv
            return carry

        lax.fori_loop(0, SEQ, row, 0)

    issue_idx(0, 0)
    issue_idx(1, 1)
    drain_idx(0, 0)
    issue_gather(0, 0)

    def outer(i, carry):
        for b in range(2):
            g = i * 2 + b
            drain_gather(g, b)

            @pl.when(g + 2 < NG)
            def _():
                issue_idx(g + 2, b)

            bo = 1 - b

            @pl.when(g + 1 < NG)
            def _():
                drain_idx(g + 1, bo)
                issue_gather(g + 1, bo)

            @pl.when(g >= 2)
            def _():
                drain_scatter(g - 2, b)

            compute(b)
            issue_scatter(g, b)
        return carry

    lax.fori_loop(0, NG // 2, outer, 0)
    drain_scatter(NG - 2, 0)
    drain_scatter(NG - 1, 1)


@jax.jit
def _emb_lookup(src, token_table, pos_table):
    fn = pl.kernel(
        _body,
        out_type=jax.ShapeDtypeStruct((BATCH, SEQ, EMB), jnp.float32),
        mesh=plsc.VectorSubcoreMesh(core_axis_name="c", subcore_axis_name="s"),
        compiler_params=pltpu.CompilerParams(use_tc_tiling_on_sc=False),
        scratch_types=[
            pltpu.VMEM((G, SEQ), jnp.int32),
            pltpu.VMEM((G, SEQ), jnp.int32),
            pltpu.VMEM((G, SEQ, EMB), jnp.float32),
            pltpu.VMEM((G, SEQ, EMB), jnp.float32),
            pltpu.VMEM((G, SEQ, EMB), jnp.float32),
            pltpu.VMEM((G, SEQ, EMB), jnp.float32),
            pltpu.VMEM((SEQ, EMB), jnp.float32),
            pltpu.SemaphoreType.DMA,
            pltpu.SemaphoreType.DMA,
            pltpu.SemaphoreType.DMA,
            pltpu.SemaphoreType.DMA,
            pltpu.SemaphoreType.DMA,
            pltpu.SemaphoreType.DMA,
        ],
    )
    return fn(src, token_table, pos_table)


def kernel(src, tgt, token_table, pos_table):
    del tgt
    return _emb_lookup(src, token_table, pos_table)


# speedup vs baseline: 1.2604x; 1.0005x over previous
import functools

import jax
import jax.numpy as jnp
from jax import lax
from jax.experimental import pallas as pl
from jax.experimental.pallas import tpu as pltpu
from jax.experimental.pallas import tpu_sc as plsc

EMB = 64
SEQ = 200
SCALE = 8.0

NC = 2
NS = 16
NW = NC * NS
LANES = 16
NREG = EMB // LANES

BATCH = 4096
B_PER_W = BATCH // NW
G = 2
NG = B_PER_W // G
HALF = SEQ // 2


def _body(src_hbm, tok_hbm, pos_hbm, out_hbm,
          ix0, ix1, ib0, ib1, ob0, ob1, pos_v,
          si0, si1, sg0, sg1, ss0, ss1):
    wid = lax.axis_index("s") * NC + lax.axis_index("c")
    row0 = wid * B_PER_W

    pltpu.sync_copy(pos_hbm, pos_v)

    ixs = (ix0, ix1)
    ibs = (ib0, ib1)
    obs = (ob0, ob1)
    si = (si0, si1)
    sg = (sg0, sg1)
    ss = (ss0, ss1)

    def issue_idx(g, b):
        for j in range(G):
            pltpu.async_copy(src_hbm.at[row0 + g * G + j], ixs[b].at[j], si[b])

    def drain_idx(g, b):
        for j in range(G):
            pltpu.make_async_copy(
                src_hbm.at[row0 + g * G + j], ixs[b].at[j], si[b]).wait()

    def issue_gather(g, b):
        del g
        for j in range(G):
            pltpu.async_copy(
                tok_hbm.at[ixs[b].at[j]], ibs[b].at[j], sg[b])

    def drain_gather(g, b):
        del g
        for j in range(G):
            pltpu.make_async_copy(
                tok_hbm.at[ixs[b].at[j]], ibs[b].at[j], sg[b]).wait()

    def issue_scatter(g, b):
        pltpu.async_copy(obs[b], out_hbm.at[pl.ds(row0 + g * G, G)], ss[b])

    def drain_scatter(g, b):
        pltpu.make_async_copy(
            obs[b], out_hbm.at[pl.ds(row0 + g * G, G)], ss[b]).wait()

    def compute(b):
        ib, ob = ibs[b], obs[b]

        def row(r, carry):
            for e in range(NREG):
                pv = pos_v[r, pl.ds(e * LANES, LANES)]
                for j in range(G):
                    tv = ib[j, r, pl.ds(e * LANES, LANES)]
                    ob[j, r, pl.ds(e * LANES, LANES)] = tv * SCALE + pv
            return carry

        lax.fori_loop(0, SEQ, row, 0)

    issue_idx(0, 0)
    issue_idx(1, 1)
    drain_idx(0, 0)
    issue_gather(0, 0)

    def outer(i, carry):
        for b in range(2):
            g = i * 2 + b
            drain_gather(g, b)

            @pl.when(g + 2 < NG)
            def _():
                issue_idx(g + 2, b)

            bo = 1 - b

            @pl.when(g + 1 < NG)
            def _():
                drain_idx(g + 1, bo)
                issue_gather(g + 1, bo)

            @pl.when(g >= 2)
            def _():
                drain_scatter(g - 2, b)

            compute(b)
            issue_scatter(g, b)
        return carry

    lax.fori_loop(0, NG // 2, outer, 0)
    drain_scatter(NG - 2, 0)
    drain_scatter(NG - 1, 1)


@jax.jit
def _emb_lookup(src, token_table, pos_table):
    fn = pl.kernel(
        _body,
        out_type=jax.ShapeDtypeStruct((BATCH, SEQ, EMB), jnp.float32),
        mesh=plsc.VectorSubcoreMesh(core_axis_name="c", subcore_axis_name="s"),
        compiler_params=pltpu.CompilerParams(
            use_tc_tiling_on_sc=False,
            disable_semaphore_checks=True,
            skip_device_barrier=True,
        ),
        scratch_types=[
            pltpu.VMEM((G, SEQ), jnp.int32),
            pltpu.VMEM((G, SEQ), jnp.int32),
            pltpu.VMEM((G, SEQ, EMB), jnp.float32),
            pltpu.VMEM((G, SEQ, EMB), jnp.float32),
            pltpu.VMEM((G, SEQ, EMB), jnp.float32),
            pltpu.VMEM((G, SEQ, EMB), jnp.float32),
            pltpu.VMEM((SEQ, EMB), jnp.float32),
            pltpu.SemaphoreType.DMA,
            pltpu.SemaphoreType.DMA,
            pltpu.SemaphoreType.DMA,
            pltpu.SemaphoreType.DMA,
            pltpu.SemaphoreType.DMA,
            pltpu.SemaphoreType.DMA,
        ],
    )
    return fn(src, token_table, pos_table)


def kernel(src, tgt, token_table, pos_table):
    del tgt
    return _emb_lookup(src, token_table, pos_table)


# speedup vs baseline: 1.2612x; 1.0006x over previous
import functools

import jax
import jax.numpy as jnp
from jax import lax
from jax.experimental import pallas as pl
from jax.experimental.pallas import tpu as pltpu
from jax.experimental.pallas import tpu_sc as plsc

EMB = 64
SEQ = 200
SCALE = 8.0

NC = 2
NS = 16
NW = NC * NS
LANES = 16
NREG = EMB // LANES

BATCH = 4096
B_PER_W = BATCH // NW
G = 2
NG = B_PER_W // G
NBUF = 4


def _body(src_hbm, tok_hbm, pos_hbm, out_hbm,
          ixs, ibs, pos_v, si, sg, ss):
    wid = lax.axis_index("s") * NC + lax.axis_index("c")
    row0 = wid * B_PER_W

    pltpu.sync_copy(pos_hbm, pos_v)

    def issue_idx(g, b):
        for j in range(G):
            pltpu.async_copy(src_hbm.at[row0 + g * G + j], ixs[b].at[j], si[b])

    def drain_idx(g, b):
        del g
        for j in range(G):
            pltpu.make_async_copy(
                src_hbm.at[row0 + j], ixs[b].at[j], si[b]).wait()

    def issue_gather(g, b):
        del g
        for j in range(G):
            pltpu.async_copy(
                tok_hbm.at[ixs[b].at[j]], ibs[b].at[j], sg[b])

    def drain_gather(g, b):
        del g
        for j in range(G):
            pltpu.make_async_copy(
                tok_hbm.at[ixs[b].at[j]], ibs[b].at[j], sg[b]).wait()

    def issue_scatter(g, b):
        pltpu.async_copy(ibs[b], out_hbm.at[pl.ds(row0 + g * G, G)], ss[b])

    def drain_scatter(g, b):
        pltpu.make_async_copy(
            ibs[b], out_hbm.at[pl.ds(row0 + g * G, G)], ss[b]).wait()

    def compute(b):
        ib = ibs[b]

        def row(r, carry):
            for e in range(NREG):
                pv = pos_v[r, pl.ds(e * LANES, LANES)]
                for j in range(G):
                    tv = ib[j, r, pl.ds(e * LANES, LANES)]
                    ib[j, r, pl.ds(e * LANES, LANES)] = tv * SCALE + pv
            return carry

        lax.fori_loop(0, SEQ, row, 0)

    for k in range(NBUF):
        issue_idx(k, k)
    for k in range(NBUF - 1):
        drain_idx(k, k)
        issue_gather(k, k)

    def outer(i, carry):
        for b in range(NBUF):
            g = i * NBUF + b
            drain_gather(g, b)
            compute(b)
            issue_scatter(g, b)

            @pl.when(g + NBUF < NG)
            def _():
                issue_idx(g + NBUF, b)

            bn = (b + NBUF - 1) % NBUF

            @pl.when(g + NBUF - 1 < NG)
            def _():
                @pl.when(g >= 1)
                def _():
                    drain_scatter(g - 1, bn)

                drain_idx(g + NBUF - 1, bn)
                issue_gather(g + NBUF - 1, bn)
        return carry

    lax.fori_loop(0, NG // NBUF, outer, 0)
    for k in range(NBUF):
        drain_scatter(NG - NBUF + k, (NG - NBUF + k) % NBUF)


@jax.jit
def _emb_lookup(src, token_table, pos_table):
    def body(src_hbm, tok_hbm, pos_hbm, out_hbm, *scratch):
        ixs = scratch[0:NBUF]
        ibs = scratch[NBUF:2 * NBUF]
        pos_v = scratch[2 * NBUF]
        si = scratch[2 * NBUF + 1:2 * NBUF + 1 + NBUF]
        sg = scratch[2 * NBUF + 1 + NBUF:2 * NBUF + 1 + 2 * NBUF]
        ss = scratch[2 * NBUF + 1 + 2 * NBUF:2 * NBUF + 1 + 3 * NBUF]
        _body(src_hbm, tok_hbm, pos_hbm, out_hbm, ixs, ibs, pos_v, si, sg, ss)

    fn = pl.kernel(
        body,
        out_type=jax.ShapeDtypeStruct((BATCH, SEQ, EMB), jnp.float32),
        mesh=plsc.VectorSubcoreMesh(core_axis_name="c", subcore_axis_name="s"),
        compiler_params=pltpu.CompilerParams(use_tc_tiling_on_sc=False),
        scratch_types=(
            [pltpu.VMEM((G, SEQ), jnp.int32) for _ in range(NBUF)]
            + [pltpu.VMEM((G, SEQ, EMB), jnp.float32) for _ in range(NBUF)]
            + [pltpu.VMEM((SEQ, EMB), jnp.float32)]
            + [pltpu.SemaphoreType.DMA for _ in range(3 * NBUF)]
        ),
    )
    return fn(src, token_table, pos_table)


def kernel(src, tgt, token_table, pos_table):
    del tgt
    return _emb_lookup(src, token_table, pos_table)
